# Initial kernel scaffold; baseline (speedup 1.0000x reference)
import jax
import jax.numpy as jnp
from jax.experimental import pallas as pl


def kernel(inputs, table):
    raise NotImplementedError("write your pallas kernel here")


# speedup vs baseline: 5.4954x; 5.4954x over previous
import dataclasses
import functools

import jax
import jax.numpy as jnp
from jax import lax
from jax.experimental import pallas as pl
from jax.experimental.pallas import tpu as pltpu
from jax.experimental.pallas import tpu_sc as plsc

_B = 4096
_F = 100
_D = 128
_L = 16
_NC = 2
_NS = 16
_NW = _NC * _NS
_BPW = _B // _NW
_NCH = _D // _L


def _compiler_params():
    cp = pltpu.CompilerParams()
    if "needs_layout_passes" in pltpu.CompilerParams.__dataclass_fields__:
        cp = dataclasses.replace(cp, needs_layout_passes=False)
    return cp


def _fm_sc(table, idx):
    mesh = plsc.VectorSubcoreMesh(core_axis_name="c", subcore_axis_name="s")

    @functools.partial(
        pl.kernel,
        out_type=jax.ShapeDtypeStruct((_B,), jnp.float32),
        mesh=mesh,
        compiler_params=_compiler_params(),
        scratch_types=[
            pltpu.VMEM((_BPW, _F), jnp.int32),
            pltpu.VMEM((_F, _D), jnp.float32),
            pltpu.VMEM((_BPW,), jnp.float32),
            pltpu.SemaphoreType.DMA,
        ],
    )
    def k(table_hbm, idx_hbm, out_hbm, idx_v, rows_v, res_v, sem):
        wid = lax.axis_index("c") * _NS + lax.axis_index("s")
        row0 = wid * _BPW
        pltpu.sync_copy(idx_hbm.at[pl.ds(row0, _BPW)], idx_v)
        lane = lax.iota(jnp.int32, _L)

        @pl.loop(0, _BPW // _L)
        def _(g):
            def row_body(j, res):
                r = g * _L + j
                pltpu.async_copy(
                    table_hbm.at[idx_v.at[r]], rows_v, sem
                ).wait()

                def acc_body(f, carry):
                    q = carry[_NCH]
                    new = []
                    for c in range(_NCH):
                        v = rows_v[f, pl.ds(c * _L, _L)]
                        new.append(carry[c] + v)
                        q = q + v * v
                    return tuple(new) + (q,)

                zeros = tuple(
                    jnp.zeros((_L,), jnp.float32) for _ in range(_NCH + 1)
                )
                acc = lax.fori_loop(0, _F, acc_body, zeros)
                diff = -acc[_NCH]
                for c in range(_NCH):
                    diff = diff + acc[c] * acc[c]
                tot = jnp.sum(diff) * 0.5
                return jnp.where(lane == j, tot, res)

            res = lax.fori_loop(0, _L, row_body, jnp.zeros((_L,), jnp.float32))
            res_v[pl.ds(g * _L, _L)] = res

        pltpu.sync_copy(res_v, out_hbm.at[pl.ds(row0, _BPW)])

    return k(table, idx)


def kernel(inputs, table):
    idx = inputs.astype(jnp.int32)
    out = _fm_sc(table, idx)
    return out.reshape(_B, 1)
